# baseline (speedup 1.0000x reference)
import functools

import jax
import jax.numpy as jnp
from jax import lax
from jax.experimental import pallas as pl
from jax.experimental.pallas import tpu as pltpu
from jax.experimental.pallas import tpu_sc as plsc

_NC = 2
_NS = 16
_NW = _NC * _NS
_CHUNK = 128


def _gather_body(uids_hbm, bids_hbm, utab_hbm, btab_hbm, ue_hbm, be_hbm,
                 uidx_v, bidx_v, rows_v, sem, *, b_per_w):
    wid = lax.axis_index("s") * _NC + lax.axis_index("c")
    base = wid * b_per_w
    pltpu.sync_copy(uids_hbm.at[pl.ds(base, b_per_w)], uidx_v)
    pltpu.sync_copy(bids_hbm.at[pl.ds(base, b_per_w)], bidx_v)
    for tab_hbm, idx_v, out_hbm in ((utab_hbm, uidx_v, ue_hbm),
                                    (btab_hbm, bidx_v, be_hbm)):
        copies = []
        for j in range(b_per_w // _CHUNK):
            s = pl.ds(j * _CHUNK, _CHUNK)
            copies.append(
                pltpu.async_copy(tab_hbm.at[idx_v.at[s]], rows_v.at[s], sem))
        for c in copies:
            c.wait()
        pltpu.sync_copy(rows_v, out_hbm.at[pl.ds(base, b_per_w)])


def _sc_gather(user_pids, book_pids, user_tab2, book_tab2):
    batch = user_pids.shape[0]
    width = user_tab2.shape[1]
    b_per_w = batch // _NW
    mesh = plsc.VectorSubcoreMesh(core_axis_name="c", subcore_axis_name="s")
    k = pl.kernel(
        functools.partial(_gather_body, b_per_w=b_per_w),
        out_type=[
            jax.ShapeDtypeStruct((batch, width), jnp.float32),
            jax.ShapeDtypeStruct((batch, width), jnp.float32),
        ],
        mesh=mesh,
        scratch_types=[
            pltpu.VMEM((b_per_w,), jnp.int32),
            pltpu.VMEM((b_per_w,), jnp.int32),
            pltpu.VMEM((b_per_w, width), jnp.float32),
            pltpu.SemaphoreType.DMA,
        ],
    )
    return k(user_pids, book_pids, user_tab2, book_tab2)


def _silu(x):
    return x * (1.0 / (1.0 + jnp.exp(-x)))


def _mlp_body(ue_ref, be_ref, up_ref, bp_ref, W1_ref, b1_ref, W2_ref, b2_ref,
              W3_ref, b3_ref, o_ref):
    e = ue_ref.shape[1] // 2
    up = up_ref[...] == 1
    bp = bp_ref[...] == 1
    u = jnp.where(up, ue_ref[:, e:], ue_ref[:, :e])
    v = jnp.where(bp, be_ref[:, e:], be_ref[:, :e])
    W1 = W1_ref[...]
    h = jnp.dot(u, W1[:e], preferred_element_type=jnp.float32)
    h += jnp.dot(v, W1[e:], preferred_element_type=jnp.float32)
    h += b1_ref[...]
    h = _silu(h)
    h = jnp.dot(h, W2_ref[...], preferred_element_type=jnp.float32)
    h += b2_ref[...]
    h = _silu(h)
    o = jnp.dot(h, W3_ref[...], preferred_element_type=jnp.float32)
    o += b3_ref[...]
    o_ref[...] = jnp.maximum(o, 0.0)


def _tc_mlp(ue, be, upar, bpar, W1, b1, W2, b2, W3, b3):
    batch, width = ue.shape
    blk = 2048
    grid = (batch // blk,)
    full = lambda shape: pl.BlockSpec(shape, lambda i: (0, 0))
    return pl.pallas_call(
        _mlp_body,
        grid=grid,
        in_specs=[
            pl.BlockSpec((blk, width), lambda i: (i, 0)),
            pl.BlockSpec((blk, width), lambda i: (i, 0)),
            pl.BlockSpec((blk, 1), lambda i: (i, 0)),
            pl.BlockSpec((blk, 1), lambda i: (i, 0)),
            full(W1.shape),
            full((1, b1.shape[0])),
            full(W2.shape),
            full((1, b2.shape[0])),
            full(W3.shape),
            full((1, 1)),
        ],
        out_specs=pl.BlockSpec((blk, 1), lambda i: (i, 0)),
        out_shape=jax.ShapeDtypeStruct((batch, 1), jnp.float32),
    )(ue, be, upar, bpar, W1, b1.reshape(1, -1), W2, b2.reshape(1, -1), W3,
      b3.reshape(1, 1))


def kernel(user_ids, book_ids, user_table, book_table, W1, b1, W2, b2, W3, b3):
    uids = user_ids.astype(jnp.int32)
    bids = book_ids.astype(jnp.int32)
    nrows, embed = user_table.shape
    utab2 = user_table.reshape(nrows // 2, embed * 2)
    btab2 = book_table.reshape(nrows // 2, embed * 2)
    ue, be = _sc_gather(uids >> 1, bids >> 1, utab2, btab2)
    upar = (uids & 1).reshape(-1, 1)
    bpar = (bids & 1).reshape(-1, 1)
    return _tc_mlp(ue, be, upar, bpar, W1, b1, W2, b2, W3, b3)
